# baseline (speedup 1.0000x reference)
import functools

import jax
import jax.numpy as jnp
from jax import lax
from jax.experimental import pallas as pl
from jax.experimental.pallas import tpu as pltpu
from jax.experimental.pallas import tpu_sc as plsc

L = 12
KV = 2
NSTATES = 1 << L
D = 1 << KV
S = 1 << (L - KV)
B = 1024
T_V = 64
BBC = 1024
NCH = B // BBC

NC, NS, LANES = 2, 16, 16
NW = NC * NS
RPG = LANES
GROUPS = B // (NW * RPG)


def _min4(t0, t1, t2, t3):
    m01 = jnp.minimum(t0, t1)
    a01 = jnp.where(t1 < t0, 1, 0)
    m23 = jnp.minimum(t2, t3)
    a23 = jnp.where(t3 < t2, 3, 2)
    m = jnp.minimum(m01, m23)
    a = jnp.where(m23 < m01, a23, a01)
    return m, a


def _limbs(a):
    hi = a.astype(jnp.bfloat16)
    r1 = a - hi.astype(jnp.float32)
    mid = r1.astype(jnp.bfloat16)
    lo = (r1 - mid.astype(jnp.float32)).astype(jnp.bfloat16)
    return hi, mid, lo


def _rep4(limbs, lo_lane, rep):
    mm = [jax.lax.dot_general(
        l[:, lo_lane : lo_lane + S // D], rep,
        dimension_numbers=(((1,), (0,)), ((), ())),
        preferred_element_type=jnp.float32,
    ) for l in limbs]
    return (mm[0] + mm[1]) + mm[2]


def _fwd_kernel(x_ref, lutl_ref, lut4_ref, rep_ref, pk_ref, rfin_ref, cost_ref):
    t = pl.program_id(1)
    lane = jax.lax.broadcasted_iota(jnp.int32, (BBC, T_V), 1)
    xt = jnp.sum(jnp.where(lane == t, x_ref[...], 0.0), axis=1,
                 keepdims=True)

    @pl.when(t == 0)
    def _():
        e = [(lutl_ref[p, :][None, :] - xt) ** 2 for p in range(D)]
        cost, low = _min4(e[0], e[1], e[2], e[3])
        cost_ref[...] = cost
        pk_ref[0, :, :] = low

    @pl.when(t > 0)
    def _():
        cost = cost_ref[...]
        rep = rep_ref[...]
        limbs = _limbs(cost)
        tot = []
        for d in range(D):
            crd = _rep4(limbs, 256 * d, rep)
            err = (lut4_ref[d, :][None, :] - xt) ** 2
            tot.append(crd + err)
        newcost, bd = _min4(tot[0], tot[1], tot[2], tot[3])
        g = t >> 4
        sh = 2 * (t & 15)
        prev = jnp.where(sh == 0, 0, pk_ref[g, :, :])
        pk_ref[g, :, :] = prev | (bd << sh)
        cost_ref[...] = newcost

        @pl.when(t == T_V - 1)
        def _():
            mfin = jnp.min(newcost, axis=1, keepdims=True)
            iota_s = jax.lax.broadcasted_iota(jnp.int32, (BBC, S), 1)
            rfin_ref[...] = jnp.min(
                jnp.where(newcost == mfin, iota_s, S), axis=1, keepdims=True)


def _forward(X, lutl, lut4, rep):
    return pl.pallas_call(
        _fwd_kernel,
        grid=(NCH, T_V),
        in_specs=[
            pl.BlockSpec((BBC, T_V), lambda j, t: (j, 0)),
            pl.BlockSpec((D, S), lambda j, t: (0, 0)),
            pl.BlockSpec((D, S), lambda j, t: (0, 0)),
            pl.BlockSpec((S // D, S), lambda j, t: (0, 0)),
        ],
        out_specs=[
            pl.BlockSpec((D, BBC, S), lambda j, t: (0, j, 0)),
            pl.BlockSpec((BBC, 1), lambda j, t: (j, 0)),
        ],
        out_shape=[
            jax.ShapeDtypeStruct((D, B, S), jnp.int32),
            jax.ShapeDtypeStruct((B, 1), jnp.int32),
        ],
        scratch_shapes=[pltpu.VMEM((BBC, S), jnp.float32)],
    )(X, lutl, lut4, rep)


def _backtrace_sc(pk, rfin, lutv):
    mesh = plsc.VectorSubcoreMesh(core_axis_name="c", subcore_axis_name="s")
    pk2 = pk.reshape(D, B * S)

    @functools.partial(
        pl.kernel,
        out_type=jax.ShapeDtypeStruct((B * T_V,), jnp.float32),
        mesh=mesh,
        compiler_params=pltpu.CompilerParams(needs_layout_passes=False),
        scratch_types=[
            pltpu.VMEM((D * RPG * S,), jnp.int32),
            pltpu.VMEM((RPG,), jnp.int32),
            pltpu.VMEM((NSTATES,), jnp.float32),
            pltpu.VMEM((RPG * T_V,), jnp.float32),
        ],
    )
    def _bt(pk_hbm, rfin_hbm, lut_hbm, out_hbm, pk_v, r0_v, lut_v, out_v):
        wid = lax.axis_index("s") * NC + lax.axis_index("c")
        lanes = lax.iota(jnp.int32, LANES)
        lanes_pk = lanes << (L - KV)
        lanes_out = lanes * T_V
        pltpu.sync_copy(lut_hbm, lut_v)
        for grp in range(GROUPS):
            base = wid * (RPG * GROUPS) + grp * RPG
            for g in range(D):
                pltpu.sync_copy(
                    pk_hbm.at[g, pl.ds(base * S, RPG * S)],
                    pk_v.at[pl.ds(g * RPG * S, RPG * S)])
            pltpu.sync_copy(rfin_hbm.at[pl.ds(base, RPG)], r0_v)
            r = r0_v[...]
            for i in range(T_V - 1, 0, -1):
                g, sh = i >> 4, 2 * (i & 15)
                vals = plsc.load_gather(pk_v, [g * (RPG * S) + lanes_pk + r])
                d = (vals >> sh) & 3
                h = plsc.load_gather(lut_v, [r + (d << 10)])
                plsc.store_scatter(out_v, [lanes_out + i], h)
                r = (r >> KV) + (d << 8)
            low = plsc.load_gather(pk_v, [lanes_pk + r]) & 3
            h0 = plsc.load_gather(lut_v, [(r << KV) + low])
            plsc.store_scatter(out_v, [lanes_out], h0)
            pltpu.sync_copy(out_v, out_hbm.at[pl.ds(base * T_V, RPG * T_V)])

    return _bt(pk2, rfin.reshape(B), lutv).reshape(B, T_V)


def kernel(X, lut):
    lutv = lut.reshape(-1).astype(jnp.float32)
    lutl = lutv.reshape(S, D).T
    lut4 = lutv.reshape(D, S)
    rep = (jnp.arange(S, dtype=jnp.int32)[None, :] // D
           == jnp.arange(S // D, dtype=jnp.int32)[:, None]).astype(jnp.bfloat16)

    pk, rfin = _forward(X, lutl, lut4, rep)
    return _backtrace_sc(pk, rfin, lutv)


# speedup vs baseline: 1.1194x; 1.1194x over previous
import functools

import jax
import jax.numpy as jnp
from jax import lax
from jax.experimental import pallas as pl
from jax.experimental.pallas import tpu as pltpu
from jax.experimental.pallas import tpu_sc as plsc

L = 12
KV = 2
NSTATES = 1 << L
D = 1 << KV
S = 1 << (L - KV)
B = 1024
T_V = 64
BBC = 512
NCH = B // BBC

NC, NS, LANES = 2, 16, 16
NW = NC * NS
RPG = LANES
GROUPS = B // (NW * RPG)


def _min4(t0, t1, t2, t3):
    m01 = jnp.minimum(t0, t1)
    a01 = jnp.where(t1 < t0, 1, 0)
    m23 = jnp.minimum(t2, t3)
    a23 = jnp.where(t3 < t2, 3, 2)
    m = jnp.minimum(m01, m23)
    a = jnp.where(m23 < m01, a23, a01)
    return m, a


def _limbs(a):
    hi = a.astype(jnp.bfloat16)
    r1 = a - hi.astype(jnp.float32)
    mid = r1.astype(jnp.bfloat16)
    lo = (r1 - mid.astype(jnp.float32)).astype(jnp.bfloat16)
    return hi, mid, lo


def _rep4(limbs, lo_lane, rep3):
    sl = slice(lo_lane, lo_lane + S // D)
    op = jnp.concatenate([l[:, sl] for l in limbs], axis=1)
    return jax.lax.dot_general(
        op, rep3,
        dimension_numbers=(((1,), (0,)), ((), ())),
        preferred_element_type=jnp.float32,
    )


def _fwd_kernel(x_ref, lutl_ref, lut4_ref, rep_ref, pk_ref, rfin_ref, cost_ref):
    t = pl.program_id(1)
    lane = jax.lax.broadcasted_iota(jnp.int32, (BBC, T_V), 1)
    xt = jnp.sum(jnp.where(lane == t, x_ref[...], 0.0), axis=1,
                 keepdims=True)

    @pl.when(t == 0)
    def _():
        e = [(lutl_ref[p, :][None, :] - xt) ** 2 for p in range(D)]
        cost, low = _min4(e[0], e[1], e[2], e[3])
        cost_ref[...] = cost
        pk_ref[0, :, :] = low

    @pl.when(t > 0)
    def _():
        cost = cost_ref[...]
        rep3 = rep_ref[...]
        limbs = _limbs(cost)
        newcost = bd = None
        for d in range(D):
            crd = _rep4(limbs, 256 * d, rep3)
            err = (lut4_ref[d, :][None, :] - xt) ** 2
            tot = crd + err
            if d == 0:
                newcost, bd = tot, jnp.zeros((BBC, S), jnp.int32)
            else:
                bd = jnp.where(tot < newcost, d, bd)
                newcost = jnp.minimum(newcost, tot)
        g = t >> 4
        sh = 2 * (t & 15)
        prev = jnp.where(sh == 0, 0, pk_ref[g, :, :])
        pk_ref[g, :, :] = prev | (bd << sh)
        cost_ref[...] = newcost

        @pl.when(t == T_V - 1)
        def _():
            mfin = jnp.min(newcost, axis=1, keepdims=True)
            iota_s = jax.lax.broadcasted_iota(jnp.int32, (BBC, S), 1)
            rfin_ref[...] = jnp.min(
                jnp.where(newcost == mfin, iota_s, S), axis=1, keepdims=True)


def _forward(X, lutl, lut4, rep):
    return pl.pallas_call(
        _fwd_kernel,
        grid=(NCH, T_V),
        in_specs=[
            pl.BlockSpec((BBC, T_V), lambda j, t: (j, 0)),
            pl.BlockSpec((D, S), lambda j, t: (0, 0)),
            pl.BlockSpec((D, S), lambda j, t: (0, 0)),
            pl.BlockSpec((3 * S // D, S), lambda j, t: (0, 0)),
        ],
        out_specs=[
            pl.BlockSpec((D, BBC, S), lambda j, t: (0, j, 0)),
            pl.BlockSpec((BBC, 1), lambda j, t: (j, 0)),
        ],
        out_shape=[
            jax.ShapeDtypeStruct((D, B, S), jnp.int32),
            jax.ShapeDtypeStruct((B, 1), jnp.int32),
        ],
        scratch_shapes=[pltpu.VMEM((BBC, S), jnp.float32)],
    )(X, lutl, lut4, rep)


def _backtrace_sc(pk, rfin, lutv):
    mesh = plsc.VectorSubcoreMesh(core_axis_name="c", subcore_axis_name="s")
    pk2 = pk.reshape(D, B * S)

    @functools.partial(
        pl.kernel,
        out_type=jax.ShapeDtypeStruct((B * T_V,), jnp.float32),
        mesh=mesh,
        compiler_params=pltpu.CompilerParams(needs_layout_passes=False),
        scratch_types=[
            pltpu.VMEM((D * RPG * S,), jnp.int32),
            pltpu.VMEM((RPG,), jnp.int32),
            pltpu.VMEM((NSTATES,), jnp.float32),
            pltpu.VMEM((RPG * T_V,), jnp.float32),
        ],
    )
    def _bt(pk_hbm, rfin_hbm, lut_hbm, out_hbm, pk_v, r0_v, lut_v, out_v):
        wid = lax.axis_index("s") * NC + lax.axis_index("c")
        lanes = lax.iota(jnp.int32, LANES)
        lanes_pk = lanes << (L - KV)
        lanes_out = lanes * T_V
        pltpu.sync_copy(lut_hbm, lut_v)
        for grp in range(GROUPS):
            base = wid * (RPG * GROUPS) + grp * RPG
            for g in range(D):
                pltpu.sync_copy(
                    pk_hbm.at[g, pl.ds(base * S, RPG * S)],
                    pk_v.at[pl.ds(g * RPG * S, RPG * S)])
            pltpu.sync_copy(rfin_hbm.at[pl.ds(base, RPG)], r0_v)
            r = r0_v[...]
            for i in range(T_V - 1, 0, -1):
                g, sh = i >> 4, 2 * (i & 15)
                vals = plsc.load_gather(pk_v, [g * (RPG * S) + lanes_pk + r])
                d = (vals >> sh) & 3
                h = plsc.load_gather(lut_v, [r + (d << 10)])
                plsc.store_scatter(out_v, [lanes_out + i], h)
                r = (r >> KV) + (d << 8)
            low = plsc.load_gather(pk_v, [lanes_pk + r]) & 3
            h0 = plsc.load_gather(lut_v, [(r << KV) + low])
            plsc.store_scatter(out_v, [lanes_out], h0)
            pltpu.sync_copy(out_v, out_hbm.at[pl.ds(base * T_V, RPG * T_V)])

    return _bt(pk2, rfin.reshape(B), lutv).reshape(B, T_V)


def kernel(X, lut):
    lutv = lut.reshape(-1).astype(jnp.float32)
    lutl = lutv.reshape(S, D).T
    lut4 = lutv.reshape(D, S)
    rep1 = (jnp.arange(S, dtype=jnp.int32)[None, :] // D
            == jnp.arange(S // D, dtype=jnp.int32)[:, None]).astype(jnp.bfloat16)
    rep = jnp.concatenate([rep1, rep1, rep1], axis=0)

    pk, rfin = _forward(X, lutl, lut4, rep)
    return _backtrace_sc(pk, rfin, lutv)
